# Initial kernel scaffold; baseline (speedup 1.0000x reference)
import jax
import jax.numpy as jnp
from jax.experimental import pallas as pl


def kernel(x, edge_index, bn_gamma, bn_beta, W, b):
    raise NotImplementedError("write your pallas kernel here")


# speedup vs baseline: 15.1230x; 15.1230x over previous
import functools

import jax
import jax.numpy as jnp
from jax import lax
from jax.experimental import pallas as pl
from jax.experimental.pallas import tpu as pltpu
from jax.experimental.pallas import tpu_sc as plsc

N = 10000
E = 320000
D = 128
L = 3
EPS = 1e-5

NC = 2
NS = 16
NW = NC * NS

C = 128
NCHUNK = E // C
CH_LO = NCHUNK // NW
CH_REM = NCHUNK - CH_LO * NW

ROWS_PER_TILE = 640
N_PAD = NS * ROWS_PER_TILE
LAST_TILE_ROWS = N - (NS - 1) * ROWS_PER_TILE


def _tc_fwd_body(x_ref, g_ref, b_ref, w_ref, bias_ref, hs_ref, outf_ref):
    h = x_ref[...]
    hs = h
    for layer in range(L):
        mean = jnp.mean(h, axis=0, keepdims=True)
        var = jnp.mean((h - mean) ** 2, axis=0, keepdims=True)
        h = (h - mean) / jnp.sqrt(var + EPS)
        h = h * g_ref[layer : layer + 1, :] + b_ref[layer : layer + 1, :]
        h = jnp.maximum(h, 0.0)
        if layer < L - 1:
            hs = hs + h
    hs_ref[...] = hs
    outf_ref[...] = (
        jnp.dot(h, w_ref[...], preferred_element_type=jnp.float32)
        + bias_ref[...]
    )


def _tc_out_body(outf_ref, p_ref, o_ref):
    o_ref[...] = outf_ref[...] + 0.0 * (p_ref[0] + p_ref[1])


_sc_mesh = plsc.VectorSubcoreMesh(
    core_axis_name="c", subcore_axis_name="s", num_cores=NC, num_subcores=NS
)


@functools.partial(
    pl.kernel,
    out_type=jax.ShapeDtypeStruct((NC, N, D), jnp.float32),
    mesh=_sc_mesh,
    scratch_types=[
        pltpu.VMEM((C,), jnp.int32),
        pltpu.VMEM((C,), jnp.int32),
        pltpu.VMEM((C, D), jnp.float32),
        pltpu.VMEM_SHARED((N_PAD, D), jnp.float32),
        pltpu.SemaphoreType.DMA,
    ],
)
def _sc_agg(hs_hbm, src_hbm, dst_hbm, zeros_hbm, out_hbm,
            dst_v, src_v, rows_v, acc_sh, sem):
    cid = lax.axis_index("c")
    sid = lax.axis_index("s")
    wid = sid * NC + cid

    r0 = sid * ROWS_PER_TILE
    pltpu.sync_copy(zeros_hbm, acc_sh.at[pl.ds(r0, ROWS_PER_TILE)])
    plsc.subcore_barrier()

    nch = jnp.where(wid < CH_REM, CH_LO + 1, CH_LO)

    @pl.loop(0, nch)
    def _chunk(j):
        base = (wid + j * NW) * C
        pltpu.sync_copy(dst_hbm.at[pl.ds(base, C)], dst_v)
        pltpu.sync_copy(src_hbm.at[pl.ds(base, C)], src_v)
        pltpu.async_copy(hs_hbm.at[dst_v], rows_v, sem).wait()
        pltpu.sync_copy(rows_v, acc_sh.at[src_v], add=True)

    plsc.subcore_barrier()

    @pl.when(sid < NS - 1)
    def _():
        pltpu.sync_copy(
            acc_sh.at[pl.ds(r0, ROWS_PER_TILE)],
            out_hbm.at[cid, pl.ds(r0, ROWS_PER_TILE)],
        )

    @pl.when(sid == NS - 1)
    def _():
        pltpu.sync_copy(
            acc_sh.at[pl.ds(r0, LAST_TILE_ROWS)],
            out_hbm.at[cid, pl.ds(r0, LAST_TILE_ROWS)],
        )


def kernel(x, edge_index, bn_gamma, bn_beta, W, b):
    src = edge_index[0]
    dst = edge_index[1]

    hs, outf = pl.pallas_call(
        _tc_fwd_body,
        out_shape=[
            jax.ShapeDtypeStruct((N, D), jnp.float32),
            jax.ShapeDtypeStruct((N, D), jnp.float32),
        ],
    )(x, bn_gamma, bn_beta, W, b.reshape(1, D))

    zeros = jnp.zeros((ROWS_PER_TILE, D), jnp.float32)
    partials = _sc_agg(hs, src, dst, zeros)

    out = pl.pallas_call(
        _tc_out_body,
        out_shape=jax.ShapeDtypeStruct((N, D), jnp.float32),
    )(outf, partials)
    return out


# speedup vs baseline: 28.0098x; 1.8521x over previous
import functools

import jax
import jax.numpy as jnp
from jax import lax
from jax.experimental import pallas as pl
from jax.experimental.pallas import tpu as pltpu
from jax.experimental.pallas import tpu_sc as plsc

N = 10000
E = 320000
D = 128
L = 3
EPS = 1e-5

NC = 2
NS = 16
NW = NC * NS

C = 128
NCH = 80
NCHUNK = NW * NCH
EP = NCHUNK * C

ROWS_PER_TILE = 640
N_PAD = NS * ROWS_PER_TILE
LAST_TILE_ROWS = N - (NS - 1) * ROWS_PER_TILE


def _tc_fwd_body(x_ref, g_ref, b_ref, w_ref, bias_ref, hs_ref, outf_ref):
    h = x_ref[...]
    hs = h
    for layer in range(L):
        mean = jnp.mean(h, axis=0, keepdims=True)
        var = jnp.mean((h - mean) ** 2, axis=0, keepdims=True)
        h = (h - mean) / jnp.sqrt(var + EPS)
        h = h * g_ref[layer : layer + 1, :] + b_ref[layer : layer + 1, :]
        h = jnp.maximum(h, 0.0)
        if layer < L - 1:
            hs = hs + h
    hs_ref[...] = hs
    outf_ref[...] = (
        jnp.dot(h, w_ref[...], preferred_element_type=jnp.float32)
        + bias_ref[...]
    )


def _tc_out_body(outf_ref, p_ref, o_ref):
    o_ref[...] = outf_ref[...] + 0.0 * (p_ref[0] + p_ref[1])


_sc_mesh = plsc.VectorSubcoreMesh(
    core_axis_name="c", subcore_axis_name="s", num_cores=NC, num_subcores=NS
)


@functools.partial(
    pl.kernel,
    out_type=jax.ShapeDtypeStruct((NC, N, D), jnp.float32),
    mesh=_sc_mesh,
    scratch_types=[
        [pltpu.VMEM((C,), jnp.int32) for _ in range(4)],
        [pltpu.VMEM((C,), jnp.int32) for _ in range(4)],
        [pltpu.VMEM((C, D), jnp.float32) for _ in range(2)],
        pltpu.VMEM_SHARED((N_PAD, D), jnp.float32),
        [pltpu.SemaphoreType.DMA for _ in range(4)],
        [pltpu.SemaphoreType.DMA for _ in range(2)],
        [pltpu.SemaphoreType.DMA for _ in range(2)],
    ],
)
def _sc_agg(hs_hbm, src_hbm, dst_hbm, zeros_hbm, out_hbm,
            idx_dst, idx_src, rows, acc_sh, isem, gsem, ssem):
    cid = lax.axis_index("c")
    sid = lax.axis_index("s")
    wid = sid * NC + cid

    def idx_start(c, slot):
        base = (wid * NCH + c) * C
        pltpu.async_copy(dst_hbm.at[pl.ds(base, C)], idx_dst[slot], isem[slot])
        pltpu.async_copy(src_hbm.at[pl.ds(base, C)], idx_src[slot], isem[slot])

    def idx_wait(slot):
        pltpu.make_async_copy(dst_hbm.at[pl.ds(0, C)], idx_dst[slot],
                              isem[slot]).wait()
        pltpu.make_async_copy(src_hbm.at[pl.ds(0, C)], idx_src[slot],
                              isem[slot]).wait()

    def gather_start(slot, b):
        pltpu.async_copy(hs_hbm.at[idx_dst[slot]], rows[b], gsem[b])

    def gather_wait(slot, b):
        pltpu.make_async_copy(hs_hbm.at[idx_dst[slot]], rows[b],
                              gsem[b]).wait()

    def scat_start(slot, b):
        pltpu.async_copy(rows[b], acc_sh.at[idx_src[slot]], ssem[b], add=True)

    def scat_wait(slot, b):
        pltpu.make_async_copy(rows[b], acc_sh.at[idx_src[slot]],
                              ssem[b]).wait()

    r0 = sid * ROWS_PER_TILE
    pltpu.sync_copy(zeros_hbm, acc_sh.at[pl.ds(r0, ROWS_PER_TILE)])
    idx_start(0, 0)
    idx_start(1, 1)
    plsc.subcore_barrier()

    @pl.loop(0, NCH, step=4)
    def _chunks(j):
        for b in range(4):
            c = j + b
            rb = b % 2

            @pl.when(c >= 2)
            def _():
                scat_wait((b - 2) % 4, rb)

            @pl.when(c + 2 < NCH)
            def _():
                idx_start(c + 2, (b + 2) % 4)

            idx_wait(b)
            gather_start(b, rb)

            @pl.when(c >= 1)
            def _():
                gather_wait((b - 1) % 4, 1 - rb)
                scat_start((b - 1) % 4, 1 - rb)

    gather_wait((NCH - 1) % 4, (NCH - 1) % 2)
    scat_start((NCH - 1) % 4, (NCH - 1) % 2)
    scat_wait((NCH - 2) % 4, (NCH - 2) % 2)
    scat_wait((NCH - 1) % 4, (NCH - 1) % 2)

    plsc.subcore_barrier()

    @pl.when(sid < NS - 1)
    def _():
        pltpu.sync_copy(
            acc_sh.at[pl.ds(r0, ROWS_PER_TILE)],
            out_hbm.at[cid, pl.ds(r0, ROWS_PER_TILE)],
        )

    @pl.when(sid == NS - 1)
    def _():
        pltpu.sync_copy(
            acc_sh.at[pl.ds(r0, LAST_TILE_ROWS)],
            out_hbm.at[cid, pl.ds(r0, LAST_TILE_ROWS)],
        )


def kernel(x, edge_index, bn_gamma, bn_beta, W, b):
    npad = EP - E
    pad_dst = (jnp.arange(npad, dtype=jnp.int32) * 131) % N
    pad_src = N + (jnp.arange(npad, dtype=jnp.int32) % (N_PAD - N))
    src = jnp.concatenate([edge_index[0], pad_src])
    dst = jnp.concatenate([edge_index[1], pad_dst])

    hs, outf = pl.pallas_call(
        _tc_fwd_body,
        out_shape=[
            jax.ShapeDtypeStruct((N, D), jnp.float32),
            jax.ShapeDtypeStruct((N, D), jnp.float32),
        ],
    )(x, bn_gamma, bn_beta, W, b.reshape(1, D))

    zeros = jnp.zeros((ROWS_PER_TILE, D), jnp.float32)
    partials = _sc_agg(hs, src, dst, zeros)

    out = pl.pallas_call(
        _tc_out_body,
        out_shape=jax.ShapeDtypeStruct((N, D), jnp.float32),
    )(outf, partials)
    return out


# speedup vs baseline: 29.0029x; 1.0355x over previous
import functools

import jax
import jax.numpy as jnp
from jax import lax
from jax.experimental import pallas as pl
from jax.experimental.pallas import tpu as pltpu
from jax.experimental.pallas import tpu_sc as plsc

N = 10000
E = 320000
D = 128
L = 3
EPS = 1e-5

NC = 2
NS = 16
NW = NC * NS

C = 112
NCH = 90
NCHUNK = NW * NCH
EP = NCHUNK * C
NZ = 64

ROWS_PER_TILE = 632
LAST_TILE_ROWS = N - (NS - 1) * ROWS_PER_TILE


def _tc_fwd_body(x_ref, g_ref, b_ref, w_ref, bias_ref, hs_ref, outf_ref):
    h = x_ref[...]
    hs = h
    for layer in range(L):
        mean = jnp.mean(h, axis=0, keepdims=True)
        var = jnp.mean((h - mean) ** 2, axis=0, keepdims=True)
        h = (h - mean) / jnp.sqrt(var + EPS)
        h = h * g_ref[layer : layer + 1, :] + b_ref[layer : layer + 1, :]
        h = jnp.maximum(h, 0.0)
        if layer < L - 1:
            hs = hs + h
    hs_ref[: N, :] = hs
    hs_ref[N :, :] = jnp.zeros((NZ, D), jnp.float32)
    outf_ref[...] = (
        jnp.dot(h, w_ref[...], preferred_element_type=jnp.float32)
        + bias_ref[...]
    )


def _tc_out_body(outf_ref, p_ref, o_ref):
    o_ref[...] = outf_ref[...] + 0.0 * (p_ref[0] + p_ref[1])


_sc_mesh = plsc.VectorSubcoreMesh(
    core_axis_name="c", subcore_axis_name="s", num_cores=NC, num_subcores=NS
)


@functools.partial(
    pl.kernel,
    out_type=jax.ShapeDtypeStruct((NC, N, D), jnp.float32),
    mesh=_sc_mesh,
    scratch_types=[
        [pltpu.VMEM((C,), jnp.int32) for _ in range(6)],
        [pltpu.VMEM((C,), jnp.int32) for _ in range(6)],
        [pltpu.VMEM((C, D), jnp.float32) for _ in range(3)],
        pltpu.VMEM_SHARED((N, D), jnp.float32),
        [pltpu.SemaphoreType.DMA for _ in range(6)],
        [pltpu.SemaphoreType.DMA for _ in range(3)],
        [pltpu.SemaphoreType.DMA for _ in range(3)],
    ],
)
def _sc_agg(hs_hbm, src_hbm, dst_hbm, zeros_hbm, out_hbm,
            idx_dst, idx_src, rows, acc_sh, isem, gsem, ssem):
    cid = lax.axis_index("c")
    sid = lax.axis_index("s")
    wid = sid * NC + cid

    def idx_start(c, slot):
        base = (wid * NCH + c) * C
        pltpu.async_copy(dst_hbm.at[pl.ds(base, C)], idx_dst[slot], isem[slot])
        pltpu.async_copy(src_hbm.at[pl.ds(base, C)], idx_src[slot], isem[slot])

    def idx_wait(slot):
        pltpu.make_async_copy(dst_hbm.at[pl.ds(0, C)], idx_dst[slot],
                              isem[slot]).wait()
        pltpu.make_async_copy(src_hbm.at[pl.ds(0, C)], idx_src[slot],
                              isem[slot]).wait()

    def gather_start(slot, b):
        pltpu.async_copy(hs_hbm.at[idx_dst[slot]], rows[b], gsem[b])

    def gather_wait(slot, b):
        pltpu.make_async_copy(hs_hbm.at[idx_dst[slot]], rows[b],
                              gsem[b]).wait()

    def scat_start(slot, b):
        pltpu.async_copy(rows[b], acc_sh.at[idx_src[slot]], ssem[b], add=True)

    def scat_wait(slot, b):
        pltpu.make_async_copy(rows[b], acc_sh.at[idx_src[slot]],
                              ssem[b]).wait()

    r0 = sid * ROWS_PER_TILE

    @pl.when(sid < NS - 1)
    def _():
        pltpu.sync_copy(zeros_hbm, acc_sh.at[pl.ds(r0, ROWS_PER_TILE)])

    @pl.when(sid == NS - 1)
    def _():
        pltpu.sync_copy(zeros_hbm.at[pl.ds(0, LAST_TILE_ROWS)],
                        acc_sh.at[pl.ds(r0, LAST_TILE_ROWS)])

    idx_start(0, 0)
    idx_start(1, 1)
    idx_start(2, 2)
    plsc.subcore_barrier()

    @pl.loop(0, NCH, step=6)
    def _chunks(j):
        for b in range(6):
            c = j + b
            rb = b % 3

            @pl.when(c >= 3)
            def _():
                scat_wait((b - 3) % 6, rb)

            @pl.when(c + 3 < NCH)
            def _():
                idx_start(c + 3, (b + 3) % 6)

            idx_wait(b)
            gather_start(b, rb)

            @pl.when(c >= 1)
            def _():
                gather_wait((b - 1) % 6, (b - 1) % 3)
                scat_start((b - 1) % 6, (b - 1) % 3)

    gather_wait((NCH - 1) % 6, (NCH - 1) % 3)
    scat_start((NCH - 1) % 6, (NCH - 1) % 3)
    scat_wait((NCH - 3) % 6, (NCH - 3) % 3)
    scat_wait((NCH - 2) % 6, (NCH - 2) % 3)
    scat_wait((NCH - 1) % 6, (NCH - 1) % 3)

    plsc.subcore_barrier()

    @pl.when(sid < NS - 1)
    def _():
        pltpu.sync_copy(
            acc_sh.at[pl.ds(r0, ROWS_PER_TILE)],
            out_hbm.at[cid, pl.ds(r0, ROWS_PER_TILE)],
        )

    @pl.when(sid == NS - 1)
    def _():
        pltpu.sync_copy(
            acc_sh.at[pl.ds(r0, LAST_TILE_ROWS)],
            out_hbm.at[cid, pl.ds(r0, LAST_TILE_ROWS)],
        )


def kernel(x, edge_index, bn_gamma, bn_beta, W, b):
    npad = EP - E
    pad_dst = N + (jnp.arange(npad, dtype=jnp.int32) % NZ)
    pad_src = (jnp.arange(npad, dtype=jnp.int32) * 131) % N
    src = jnp.concatenate([edge_index[0], pad_src])
    dst = jnp.concatenate([edge_index[1], pad_dst])

    hs, outf = pl.pallas_call(
        _tc_fwd_body,
        out_shape=[
            jax.ShapeDtypeStruct((N + NZ, D), jnp.float32),
            jax.ShapeDtypeStruct((N, D), jnp.float32),
        ],
    )(x, bn_gamma, bn_beta, W, b.reshape(1, D))

    zeros = jnp.zeros((ROWS_PER_TILE, D), jnp.float32)
    partials = _sc_agg(hs, src, dst, zeros)

    out = pl.pallas_call(
        _tc_out_body,
        out_shape=jax.ShapeDtypeStruct((N, D), jnp.float32),
    )(outf, partials)
    return out
